# baseline (speedup 1.0000x reference)
import functools

import jax
import jax.numpy as jnp
from jax import lax
from jax.experimental import pallas as pl
from jax.experimental.pallas import tpu as pltpu
from jax.experimental.pallas import tpu_sc as plsc

_N = 10000
_NP = 10240
_E = 320000
_EP = 327680
_TCH = 2560
_CW = 128
_G = 32
_RPT = _NP // 16
_K0 = 152
_K1 = 8


def _segsum(table, col0, row0, col1, row1, feat):
    mesh = plsc.VectorSubcoreMesh(core_axis_name="c", subcore_axis_name="s")

    @functools.partial(
        pl.kernel,
        out_type=jax.ShapeDtypeStruct((2, _NP, feat), jnp.float32),
        mesh=mesh,
        compiler_params=pltpu.CompilerParams(use_tc_tiling_on_sc=False),
        scratch_types=[
            pltpu.VMEM((_K0, _CW), jnp.int32),
            pltpu.VMEM((_K0, _CW), jnp.int32),
            pltpu.VMEM((4, _CW, feat), jnp.float32),
            pltpu.VMEM_SHARED((_NP, feat), jnp.float32),
            pltpu.SemaphoreType.DMA,
            pltpu.SemaphoreType.DMA,
        ],
    )
    def seg(table_hbm, col0_hbm, row0_hbm, col1_hbm, row1_hbm,
            out_hbm, col_v, row_v, gbuf, acc, sem_g, sem_s):
        c = lax.axis_index("c")
        s = lax.axis_index("s")

        def zrow(i, carry):
            for j in range(feat // 16):
                gbuf[0, i, pl.ds(j * 16, 16)] = jnp.zeros((16,), jnp.float32)
            return carry

        lax.fori_loop(0, _CW, zrow, 0)
        for k in range(_RPT // _CW):
            pltpu.sync_copy(gbuf.at[0], acc.at[pl.ds(s * _RPT + k * _CW, _CW)])

        def body(t, carry):
            base = t * 4
            gets = [pltpu.async_copy(table_hbm.at[col_v.at[base + b]],
                                     gbuf.at[b], sem_g)
                    for b in range(4)]
            for g in gets:
                g.wait()
            for b in range(4):
                pltpu.async_copy(gbuf.at[b], acc.at[row_v.at[base + b]],
                                 sem_s, add=True).wait()
            return carry

        @pl.when(c == 0)
        def _():
            pltpu.sync_copy(col0_hbm.at[s], col_v)
            pltpu.sync_copy(row0_hbm.at[s], row_v)
            plsc.subcore_barrier()
            lax.fori_loop(0, _K0 // 4, body, 0)

        @pl.when(c == 1)
        def _():
            pltpu.sync_copy(col1_hbm.at[s], col_v.at[pl.ds(0, _K1)])
            pltpu.sync_copy(row1_hbm.at[s], row_v.at[pl.ds(0, _K1)])
            plsc.subcore_barrier()
            lax.fori_loop(0, _K1 // 4, body, 0)

        plsc.subcore_barrier()
        pltpu.sync_copy(acc.at[pl.ds(s * _RPT, _RPT)],
                        out_hbm.at[c, pl.ds(s * _RPT, _RPT)])

    return seg(table, col0, row0, col1, row1)


def _proj1_body(x_ref, w_ref, o_ref):
    o_ref[...] = jnp.dot(x_ref[...], w_ref[...],
                         preferred_element_type=jnp.float32)


def _proj2_body(s_ref, w_ref, o_ref):
    h = jnp.maximum(s_ref[0] + s_ref[1], 0.0)
    o_ref[...] = jnp.dot(h, w_ref[...], preferred_element_type=jnp.float32)


def _readout_body(s_ref, b_ref, w1_ref, b1_ref, w2_ref, b2_ref, o_ref,
                  acc, cnt):
    i = pl.program_id(0)

    @pl.when(i == 0)
    def _():
        acc[...] = jnp.zeros_like(acc)
        cnt[...] = jnp.zeros_like(cnt)

    h = jnp.maximum(s_ref[0] + s_ref[1], 0.0)
    b = b_ref[0]
    onehot = (b == lax.broadcasted_iota(jnp.int32, (1280, _G), 1)
              ).astype(jnp.float32)
    dn = (((0,), (0,)), ((), ()))
    acc[...] += lax.dot_general(onehot, h, dn,
                                preferred_element_type=jnp.float32)
    cnt[...] += lax.dot_general(onehot, jnp.ones((1280, 64), jnp.float32),
                                dn, preferred_element_type=jnp.float32)

    @pl.when(i == pl.num_programs(0) - 1)
    def _():
        g = acc[...] / jnp.maximum(cnt[...], 1.0)
        t = jnp.maximum(
            jnp.dot(g, w1_ref[...], preferred_element_type=jnp.float32)
            + b1_ref[...], 0.0)
        o_ref[...] = (jnp.dot(t, w2_ref[...],
                              preferred_element_type=jnp.float32)
                      + b2_ref[...])


def kernel(x, edge_attr, c1_fc, c1_ed, c1_att, c2_fc, c2_ed, c2_att,
           e1_fc, e1_ed, e1_att, e2_fc, e2_ed, e2_att,
           fc1_w, fc1_b, fc2_w, fc2_b, edge_index, batch):
    f32 = jnp.float32
    xpad = jnp.concatenate([x, jnp.zeros((_NP - _N, x.shape[1]), f32)], axis=0)
    w1 = jnp.concatenate([c1_fc.T, e1_fc.T], axis=1)
    w2 = jnp.zeros((32, 64), f32)
    w2 = w2.at[:16, :32].set(c2_fc.T).at[16:, 32:].set(e2_fc.T)

    pad = _EP - _E
    colp = jnp.concatenate([edge_index[1], jnp.full((pad,), _N, jnp.int32)])
    rowp = jnp.concatenate([edge_index[0], jnp.zeros((pad,), jnp.int32)])
    split = 16 * _K0 * _CW
    col0 = colp[:split].reshape(16, _K0, _CW)
    row0 = rowp[:split].reshape(16, _K0, _CW)
    col1 = colp[split:].reshape(16, _K1, _CW)
    row1 = rowp[split:].reshape(16, _K1, _CW)

    batchp = jnp.concatenate([batch, jnp.full((_NP - _N,), _G, jnp.int32)])
    batch3 = batchp.reshape(8, 1280, 1)

    xp = pl.pallas_call(
        _proj1_body,
        grid=(8,),
        in_specs=[pl.BlockSpec((1280, 128), lambda i: (i, 0)),
                  pl.BlockSpec((128, 32), lambda i: (0, 0))],
        out_specs=pl.BlockSpec((1280, 32), lambda i: (i, 0)),
        out_shape=jax.ShapeDtypeStruct((_NP, 32), f32),
    )(xpad, w1)

    s1p = _segsum(xp, col0, row0, col1, row1, 32)

    zp = pl.pallas_call(
        _proj2_body,
        grid=(8,),
        in_specs=[pl.BlockSpec((2, 1280, 32), lambda i: (0, i, 0)),
                  pl.BlockSpec((32, 64), lambda i: (0, 0))],
        out_specs=pl.BlockSpec((1280, 64), lambda i: (i, 0)),
        out_shape=jax.ShapeDtypeStruct((_NP, 64), f32),
    )(s1p, w2)

    s2p = _segsum(zp, col0, row0, col1, row1, 64)

    out = pl.pallas_call(
        _readout_body,
        grid=(8,),
        in_specs=[pl.BlockSpec((2, 1280, 64), lambda i: (0, i, 0)),
                  pl.BlockSpec((1, 1280, 1), lambda i: (i, 0, 0)),
                  pl.BlockSpec((64, 128), lambda i: (0, 0)),
                  pl.BlockSpec((1, 128), lambda i: (0, 0)),
                  pl.BlockSpec((128, 1), lambda i: (0, 0)),
                  pl.BlockSpec((1, 1), lambda i: (0, 0))],
        out_specs=pl.BlockSpec((_G, 1), lambda i: (0, 0)),
        out_shape=jax.ShapeDtypeStruct((_G, 1), f32),
        scratch_shapes=[pltpu.VMEM((_G, 64), f32), pltpu.VMEM((_G, 64), f32)],
    )(s2p, batch3, fc1_w.T, fc1_b.reshape(1, 128), fc2_w.T,
      fc2_b.reshape(1, 1))
    return out


# speedup vs baseline: 1.0404x; 1.0404x over previous
import functools

import jax
import jax.numpy as jnp
from jax import lax
from jax.experimental import pallas as pl
from jax.experimental.pallas import tpu as pltpu
from jax.experimental.pallas import tpu_sc as plsc

_N = 10000
_NP = 10240
_E = 320000
_EP = 327680
_TCH = 2560
_CW = 128
_G = 32
_RPT = _NP // 16
_K0 = 140
_K1 = 20


def _segsum(table, col0, row0, col1, row1, feat):
    mesh = plsc.VectorSubcoreMesh(core_axis_name="c", subcore_axis_name="s")

    @functools.partial(
        pl.kernel,
        out_type=jax.ShapeDtypeStruct((2, _NP, feat), jnp.float32),
        mesh=mesh,
        compiler_params=pltpu.CompilerParams(use_tc_tiling_on_sc=False),
        scratch_types=[
            pltpu.VMEM((_K0, _CW), jnp.int32),
            pltpu.VMEM((_K0, _CW), jnp.int32),
            pltpu.VMEM((4, _CW, feat), jnp.float32),
            pltpu.VMEM_SHARED((_NP, feat), jnp.float32),
            pltpu.SemaphoreType.DMA,
            pltpu.SemaphoreType.DMA,
        ],
    )
    def seg(table_hbm, col0_hbm, row0_hbm, col1_hbm, row1_hbm,
            out_hbm, col_v, row_v, gbuf, acc, sem_g, sem_s):
        c = lax.axis_index("c")
        s = lax.axis_index("s")

        def zrow(i, carry):
            for j in range(feat // 16):
                gbuf[0, i, pl.ds(j * 16, 16)] = jnp.zeros((16,), jnp.float32)
            return carry

        lax.fori_loop(0, _CW, zrow, 0)
        for k in range(_RPT // _CW):
            pltpu.sync_copy(gbuf.at[0], acc.at[pl.ds(s * _RPT + k * _CW, _CW)])

        def body(t, carry):
            base = t * 4
            gets = [pltpu.async_copy(table_hbm.at[col_v.at[base + b]],
                                     gbuf.at[b], sem_g)
                    for b in range(4)]
            for g in gets:
                g.wait()
            for b in range(4):
                pltpu.async_copy(gbuf.at[b], acc.at[row_v.at[base + b]],
                                 sem_s, add=True).wait()
            return carry

        @pl.when(c == 0)
        def _():
            pltpu.sync_copy(col0_hbm.at[s], col_v)
            pltpu.sync_copy(row0_hbm.at[s], row_v)
            plsc.subcore_barrier()
            lax.fori_loop(0, _K0 // 4, body, 0)

        @pl.when(c == 1)
        def _():
            pltpu.sync_copy(col1_hbm.at[s], col_v.at[pl.ds(0, _K1)])
            pltpu.sync_copy(row1_hbm.at[s], row_v.at[pl.ds(0, _K1)])
            plsc.subcore_barrier()
            lax.fori_loop(0, _K1 // 4, body, 0)

        plsc.subcore_barrier()
        pltpu.sync_copy(acc.at[pl.ds(s * _RPT, _RPT)],
                        out_hbm.at[c, pl.ds(s * _RPT, _RPT)])

    return seg(table, col0, row0, col1, row1)


def _proj1_body(x_ref, w_ref, o_ref):
    o_ref[...] = jnp.dot(x_ref[...], w_ref[...],
                         preferred_element_type=jnp.float32)


def _proj2_body(s_ref, w_ref, o_ref):
    h = jnp.maximum(s_ref[0] + s_ref[1], 0.0)
    o_ref[...] = jnp.dot(h, w_ref[...], preferred_element_type=jnp.float32)


def _readout_body(s_ref, b_ref, w1_ref, b1_ref, w2_ref, b2_ref, o_ref,
                  acc, cnt):
    i = pl.program_id(0)

    @pl.when(i == 0)
    def _():
        acc[...] = jnp.zeros_like(acc)
        cnt[...] = jnp.zeros_like(cnt)

    h = jnp.maximum(s_ref[0] + s_ref[1], 0.0)
    b = b_ref[0]
    onehot = (b == lax.broadcasted_iota(jnp.int32, (1280, _G), 1)
              ).astype(jnp.float32)
    dn = (((0,), (0,)), ((), ()))
    acc[...] += lax.dot_general(onehot, h, dn,
                                preferred_element_type=jnp.float32)
    cnt[...] += lax.dot_general(onehot, jnp.ones((1280, 64), jnp.float32),
                                dn, preferred_element_type=jnp.float32)

    @pl.when(i == pl.num_programs(0) - 1)
    def _():
        g = acc[...] / jnp.maximum(cnt[...], 1.0)
        t = jnp.maximum(
            jnp.dot(g, w1_ref[...], preferred_element_type=jnp.float32)
            + b1_ref[...], 0.0)
        o_ref[...] = (jnp.dot(t, w2_ref[...],
                              preferred_element_type=jnp.float32)
                      + b2_ref[...])


def kernel(x, edge_attr, c1_fc, c1_ed, c1_att, c2_fc, c2_ed, c2_att,
           e1_fc, e1_ed, e1_att, e2_fc, e2_ed, e2_att,
           fc1_w, fc1_b, fc2_w, fc2_b, edge_index, batch):
    f32 = jnp.float32
    xpad = jnp.concatenate([x, jnp.zeros((_NP - _N, x.shape[1]), f32)], axis=0)
    w1 = jnp.concatenate([c1_fc.T, e1_fc.T], axis=1)
    w2 = jnp.zeros((32, 64), f32)
    w2 = w2.at[:16, :32].set(c2_fc.T).at[16:, 32:].set(e2_fc.T)

    pad = _EP - _E
    colp = jnp.concatenate([edge_index[1], jnp.full((pad,), _N, jnp.int32)])
    rowp = jnp.concatenate([edge_index[0], jnp.zeros((pad,), jnp.int32)])
    split = 16 * _K0 * _CW
    col0 = colp[:split].reshape(16, _K0, _CW)
    row0 = rowp[:split].reshape(16, _K0, _CW)
    col1 = colp[split:].reshape(16, _K1, _CW)
    row1 = rowp[split:].reshape(16, _K1, _CW)

    batchp = jnp.concatenate([batch, jnp.full((_NP - _N,), _G, jnp.int32)])
    batch3 = batchp.reshape(8, 1280, 1)

    xp = pl.pallas_call(
        _proj1_body,
        grid=(8,),
        in_specs=[pl.BlockSpec((1280, 128), lambda i: (i, 0)),
                  pl.BlockSpec((128, 32), lambda i: (0, 0))],
        out_specs=pl.BlockSpec((1280, 32), lambda i: (i, 0)),
        out_shape=jax.ShapeDtypeStruct((_NP, 32), f32),
    )(xpad, w1)

    s1p = _segsum(xp, col0, row0, col1, row1, 32)

    zp = pl.pallas_call(
        _proj2_body,
        grid=(8,),
        in_specs=[pl.BlockSpec((2, 1280, 32), lambda i: (0, i, 0)),
                  pl.BlockSpec((32, 64), lambda i: (0, 0))],
        out_specs=pl.BlockSpec((1280, 64), lambda i: (i, 0)),
        out_shape=jax.ShapeDtypeStruct((_NP, 64), f32),
    )(s1p, w2)

    s2p = _segsum(zp, col0, row0, col1, row1, 64)

    out = pl.pallas_call(
        _readout_body,
        grid=(8,),
        in_specs=[pl.BlockSpec((2, 1280, 64), lambda i: (0, i, 0)),
                  pl.BlockSpec((1, 1280, 1), lambda i: (i, 0, 0)),
                  pl.BlockSpec((64, 128), lambda i: (0, 0)),
                  pl.BlockSpec((1, 128), lambda i: (0, 0)),
                  pl.BlockSpec((128, 1), lambda i: (0, 0)),
                  pl.BlockSpec((1, 1), lambda i: (0, 0))],
        out_specs=pl.BlockSpec((_G, 1), lambda i: (0, 0)),
        out_shape=jax.ShapeDtypeStruct((_G, 1), f32),
        scratch_shapes=[pltpu.VMEM((_G, 64), f32), pltpu.VMEM((_G, 64), f32)],
    )(s2p, batch3, fc1_w.T, fc1_b.reshape(1, 128), fc2_w.T,
      fc2_b.reshape(1, 1))
    return out


# speedup vs baseline: 1.0472x; 1.0065x over previous
import functools

import jax
import jax.numpy as jnp
from jax import lax
from jax.experimental import pallas as pl
from jax.experimental.pallas import tpu as pltpu
from jax.experimental.pallas import tpu_sc as plsc

_N = 10000
_NP = 10240
_E = 320000
_EP = 327680
_TCH = 2560
_CW = 128
_G = 32
_RPT = _NP // 16
_K0 = 144
_K1 = 16


def _segsum(table, col0, row0, col1, row1, feat):
    mesh = plsc.VectorSubcoreMesh(core_axis_name="c", subcore_axis_name="s")

    @functools.partial(
        pl.kernel,
        out_type=jax.ShapeDtypeStruct((2, _NP, feat), jnp.float32),
        mesh=mesh,
        compiler_params=pltpu.CompilerParams(use_tc_tiling_on_sc=False),
        scratch_types=[
            pltpu.VMEM((_K0, _CW), jnp.int32),
            pltpu.VMEM((_K0, _CW), jnp.int32),
            pltpu.VMEM((4, _CW, feat), jnp.float32),
            pltpu.VMEM_SHARED((_NP, feat), jnp.float32),
            pltpu.SemaphoreType.DMA,
            pltpu.SemaphoreType.DMA,
        ],
    )
    def seg(table_hbm, col0_hbm, row0_hbm, col1_hbm, row1_hbm,
            out_hbm, col_v, row_v, gbuf, acc, sem_g, sem_s):
        c = lax.axis_index("c")
        s = lax.axis_index("s")

        def zrow(i, carry):
            for j in range(feat // 16):
                gbuf[0, i, pl.ds(j * 16, 16)] = jnp.zeros((16,), jnp.float32)
            return carry

        lax.fori_loop(0, _CW, zrow, 0)
        for k in range(_RPT // _CW):
            pltpu.sync_copy(gbuf.at[0], acc.at[pl.ds(s * _RPT + k * _CW, _CW)])

        def body(t, carry):
            base = t * 4
            gets = [pltpu.async_copy(table_hbm.at[col_v.at[base + b]],
                                     gbuf.at[b], sem_g)
                    for b in range(4)]
            for g in gets:
                g.wait()
            for b in range(4):
                pltpu.async_copy(gbuf.at[b], acc.at[row_v.at[base + b]],
                                 sem_s, add=True).wait()
            return carry

        @pl.when(c == 0)
        def _():
            pltpu.sync_copy(col0_hbm.at[s], col_v)
            pltpu.sync_copy(row0_hbm.at[s], row_v)
            plsc.subcore_barrier()
            lax.fori_loop(0, _K0 // 4, body, 0)

        @pl.when(c == 1)
        def _():
            pltpu.sync_copy(col1_hbm.at[s], col_v.at[pl.ds(0, _K1)])
            pltpu.sync_copy(row1_hbm.at[s], row_v.at[pl.ds(0, _K1)])
            plsc.subcore_barrier()
            lax.fori_loop(0, _K1 // 4, body, 0)

        plsc.subcore_barrier()
        pltpu.sync_copy(acc.at[pl.ds(s * _RPT, _RPT)],
                        out_hbm.at[c, pl.ds(s * _RPT, _RPT)])

    return seg(table, col0, row0, col1, row1)


def _proj1_body(x_ref, w_ref, o_ref):
    o_ref[...] = jnp.dot(x_ref[...], w_ref[...],
                         preferred_element_type=jnp.float32)


def _proj2_body(s_ref, w_ref, o_ref):
    h = jnp.maximum(s_ref[0] + s_ref[1], 0.0)
    o_ref[...] = jnp.dot(h, w_ref[...], preferred_element_type=jnp.float32)


def _readout_body(s_ref, b_ref, w1_ref, b1_ref, w2_ref, b2_ref, o_ref,
                  acc, cnt):
    i = pl.program_id(0)

    @pl.when(i == 0)
    def _():
        acc[...] = jnp.zeros_like(acc)
        cnt[...] = jnp.zeros_like(cnt)

    h = jnp.maximum(s_ref[0] + s_ref[1], 0.0)
    b = b_ref[0]
    onehot = (b == lax.broadcasted_iota(jnp.int32, (1280, _G), 1)
              ).astype(jnp.float32)
    dn = (((0,), (0,)), ((), ()))
    acc[...] += lax.dot_general(onehot, h, dn,
                                preferred_element_type=jnp.float32)
    cnt[...] += lax.dot_general(onehot, jnp.ones((1280, 64), jnp.float32),
                                dn, preferred_element_type=jnp.float32)

    @pl.when(i == pl.num_programs(0) - 1)
    def _():
        g = acc[...] / jnp.maximum(cnt[...], 1.0)
        t = jnp.maximum(
            jnp.dot(g, w1_ref[...], preferred_element_type=jnp.float32)
            + b1_ref[...], 0.0)
        o_ref[...] = (jnp.dot(t, w2_ref[...],
                              preferred_element_type=jnp.float32)
                      + b2_ref[...])


def kernel(x, edge_attr, c1_fc, c1_ed, c1_att, c2_fc, c2_ed, c2_att,
           e1_fc, e1_ed, e1_att, e2_fc, e2_ed, e2_att,
           fc1_w, fc1_b, fc2_w, fc2_b, edge_index, batch):
    f32 = jnp.float32
    xpad = jnp.concatenate([x, jnp.zeros((_NP - _N, x.shape[1]), f32)], axis=0)
    w1 = jnp.concatenate([c1_fc.T, e1_fc.T], axis=1)
    w2 = jnp.zeros((32, 64), f32)
    w2 = w2.at[:16, :32].set(c2_fc.T).at[16:, 32:].set(e2_fc.T)

    pad = _EP - _E
    colp = jnp.concatenate([edge_index[1], jnp.full((pad,), _N, jnp.int32)])
    rowp = jnp.concatenate([edge_index[0], jnp.zeros((pad,), jnp.int32)])
    split = 16 * _K0 * _CW
    col0 = colp[:split].reshape(16, _K0, _CW)
    row0 = rowp[:split].reshape(16, _K0, _CW)
    col1 = colp[split:].reshape(16, _K1, _CW)
    row1 = rowp[split:].reshape(16, _K1, _CW)

    batchp = jnp.concatenate([batch, jnp.full((_NP - _N,), _G, jnp.int32)])
    batch3 = batchp.reshape(8, 1280, 1)

    xp = pl.pallas_call(
        _proj1_body,
        grid=(8,),
        in_specs=[pl.BlockSpec((1280, 128), lambda i: (i, 0)),
                  pl.BlockSpec((128, 32), lambda i: (0, 0))],
        out_specs=pl.BlockSpec((1280, 32), lambda i: (i, 0)),
        out_shape=jax.ShapeDtypeStruct((_NP, 32), f32),
    )(xpad, w1)

    s1p = _segsum(xp, col0, row0, col1, row1, 32)

    zp = pl.pallas_call(
        _proj2_body,
        grid=(8,),
        in_specs=[pl.BlockSpec((2, 1280, 32), lambda i: (0, i, 0)),
                  pl.BlockSpec((32, 64), lambda i: (0, 0))],
        out_specs=pl.BlockSpec((1280, 64), lambda i: (i, 0)),
        out_shape=jax.ShapeDtypeStruct((_NP, 64), f32),
    )(s1p, w2)

    s2p = _segsum(zp, col0, row0, col1, row1, 64)

    out = pl.pallas_call(
        _readout_body,
        grid=(8,),
        in_specs=[pl.BlockSpec((2, 1280, 64), lambda i: (0, i, 0)),
                  pl.BlockSpec((1, 1280, 1), lambda i: (i, 0, 0)),
                  pl.BlockSpec((64, 128), lambda i: (0, 0)),
                  pl.BlockSpec((1, 128), lambda i: (0, 0)),
                  pl.BlockSpec((128, 1), lambda i: (0, 0)),
                  pl.BlockSpec((1, 1), lambda i: (0, 0))],
        out_specs=pl.BlockSpec((_G, 1), lambda i: (0, 0)),
        out_shape=jax.ShapeDtypeStruct((_G, 1), f32),
        scratch_shapes=[pltpu.VMEM((_G, 64), f32), pltpu.VMEM((_G, 64), f32)],
    )(s2p, batch3, fc1_w.T, fc1_b.reshape(1, 128), fc2_w.T,
      fc2_b.reshape(1, 1))
    return out
